# Initial kernel scaffold; baseline (speedup 1.0000x reference)
import jax
import jax.numpy as jnp
from jax.experimental import pallas as pl


def kernel(x, Win, b_in, Wout, b_out, codebooks):
    raise NotImplementedError("write your pallas kernel here")


# speedup vs baseline: 1.3089x; 1.3089x over previous
import jax
import jax.numpy as jnp
from jax.experimental import pallas as pl
from jax.experimental.pallas import tpu as pltpu

B, D, T = 16, 1024, 512
N_CB, CB_SIZE, CB_DIM = 8, 1024, 256
ROWS = B * T
R = 1024
N_TILES = ROWS // R
KC = 2
KW = D // KC
bf = jnp.bfloat16
f32 = jnp.float32
_DN = (((1,), (1,)), ((), ()))


def _prep_body(cb_ref, cbn_ref, ct_ref, hi_ref, lo_ref, lolo_ref):
    c = cb_ref[0]
    nrm = jnp.sqrt(jnp.sum(c * c, axis=-1, keepdims=True))
    cbn = c / (nrm + 1e-12)
    cbn_ref[0] = cbn.astype(bf)
    ct_ref[0] = jnp.sum(cbn * cbn, axis=-1)[None, :]
    hi = c.astype(bf)
    r1 = c - hi.astype(f32)
    lo = r1.astype(bf)
    r2 = r1 - lo.astype(f32)
    hi_ref[0] = hi
    lo_ref[0] = lo
    lolo_ref[0] = r2.astype(bf)


def _rvq_body(xr4_ref, win_ref, bin_ref, cbn_ref, ct_ref,
              hi_ref, lo_ref, lolo_ref, wout_ref, bout_ref,
              z4_ref, codes_ref, loss_ref, resid_ref, acc_ref, zacc_ref):
    s = pl.program_id(1)
    k = pl.program_id(2)

    @pl.when(jnp.logical_and(s == 0, k == 0))
    def _():
        resid_ref[:] = xr4_ref[:]

    part = jax.lax.dot_general(resid_ref[k].astype(bf), win_ref[0, 0], _DN,
                               preferred_element_type=f32)

    @pl.when(k == 0)
    def _():
        acc_ref[:] = part

    @pl.when(k > 0)
    def _():
        acc_ref[:] = acc_ref[:] + part

    @pl.when(k == KC - 1)
    def _():
        enc = acc_ref[:] + bin_ref[0]
        nrm = jnp.sqrt(jnp.sum(enc * enc, axis=1, keepdims=True))
        enc_n = enc / (nrm + 1e-12)
        scores = jax.lax.dot_general(enc_n.astype(bf), cbn_ref[0], _DN,
                                     preferred_element_type=f32)
        rowterm = jnp.sum(enc_n * enc_n, axis=1, keepdims=True)
        dist = (rowterm - 2.0 * scores) + ct_ref[0]
        m = jnp.min(dist, axis=1, keepdims=True)
        iota = jax.lax.broadcasted_iota(jnp.int32, (R, CB_SIZE), 1)
        idx = jnp.min(jnp.where(dist == m, iota, CB_SIZE), axis=1)
        onehot = (iota == idx[:, None]).astype(bf)
        gd = (((1,), (0,)), ((), ()))
        z_q = (jax.lax.dot_general(onehot, hi_ref[0], gd, preferred_element_type=f32)
               + jax.lax.dot_general(onehot, lo_ref[0], gd, preferred_element_type=f32)
               ) + jax.lax.dot_general(onehot, lolo_ref[0], gd,
                                       preferred_element_type=f32)
        zst_b = (enc + (z_q - enc)).astype(bf)
        for kk in range(KC):
            out_kk = jax.lax.dot_general(zst_b, wout_ref[0, kk], _DN,
                                         preferred_element_type=f32) \
                + bout_ref[0, kk]
            resid_ref[kk] = resid_ref[kk] - out_kk

            @pl.when(s == 0)
            def _(out_kk=out_kk, kk=kk):
                zacc_ref[kk] = out_kk

            @pl.when(s > 0)
            def _(out_kk=out_kk, kk=kk):
                zacc_ref[kk] = zacc_ref[kk] + out_kk

        codes_ref[pl.ds(s, 1), :] = idx[None, :]
        d = enc - z_q
        loss_ref[0, 0, 0, :] = jnp.full((128,), jnp.sum(d * d), f32)

        @pl.when(s == N_CB - 1)
        def _():
            z4_ref[:] = zacc_ref[:]


def kernel(x, Win, b_in, Wout, b_out, codebooks):
    cbn_b, ct, cb_hi, cb_lo, cb_lolo = pl.pallas_call(
        _prep_body,
        grid=(N_CB,),
        in_specs=[pl.BlockSpec((1, CB_SIZE, CB_DIM), lambda i: (i, 0, 0))],
        out_specs=[
            pl.BlockSpec((1, CB_SIZE, CB_DIM), lambda i: (i, 0, 0)),
            pl.BlockSpec((1, 1, CB_SIZE), lambda i: (i, 0, 0)),
            pl.BlockSpec((1, CB_SIZE, CB_DIM), lambda i: (i, 0, 0)),
            pl.BlockSpec((1, CB_SIZE, CB_DIM), lambda i: (i, 0, 0)),
            pl.BlockSpec((1, CB_SIZE, CB_DIM), lambda i: (i, 0, 0)),
        ],
        out_shape=[
            jax.ShapeDtypeStruct((N_CB, CB_SIZE, CB_DIM), bf),
            jax.ShapeDtypeStruct((N_CB, 1, CB_SIZE), f32),
            jax.ShapeDtypeStruct((N_CB, CB_SIZE, CB_DIM), bf),
            jax.ShapeDtypeStruct((N_CB, CB_SIZE, CB_DIM), bf),
            jax.ShapeDtypeStruct((N_CB, CB_SIZE, CB_DIM), bf),
        ],
    )(codebooks)

    xr = x.transpose(0, 2, 1).reshape(ROWS, D)
    xr4 = xr.reshape(ROWS, KC, KW).transpose(1, 0, 2)
    win4 = Win.astype(bf).reshape(N_CB, CB_DIM, KC, KW).transpose(0, 2, 1, 3)
    wout4 = Wout.astype(bf).reshape(N_CB, KC, KW, CB_DIM)
    b_in3 = b_in.reshape(N_CB, 1, CB_DIM)
    b_out4 = b_out.reshape(N_CB, KC, 1, KW)

    z4, codes, loss_parts = pl.pallas_call(
        _rvq_body,
        grid=(N_TILES, N_CB, KC),
        in_specs=[
            pl.BlockSpec((KC, R, KW), lambda t, s, k: (0, t, 0)),
            pl.BlockSpec((1, 1, CB_DIM, KW), lambda t, s, k: (s, k, 0, 0)),
            pl.BlockSpec((1, 1, CB_DIM), lambda t, s, k: (s, 0, 0)),
            pl.BlockSpec((1, CB_SIZE, CB_DIM), lambda t, s, k: (s, 0, 0)),
            pl.BlockSpec((1, 1, CB_SIZE), lambda t, s, k: (s, 0, 0)),
            pl.BlockSpec((1, CB_SIZE, CB_DIM), lambda t, s, k: (s, 0, 0)),
            pl.BlockSpec((1, CB_SIZE, CB_DIM), lambda t, s, k: (s, 0, 0)),
            pl.BlockSpec((1, CB_SIZE, CB_DIM), lambda t, s, k: (s, 0, 0)),
            pl.BlockSpec((1, KC, KW, CB_DIM), lambda t, s, k: (s, 0, 0, 0)),
            pl.BlockSpec((1, KC, 1, KW), lambda t, s, k: (s, 0, 0, 0)),
        ],
        out_specs=[
            pl.BlockSpec((KC, R, KW), lambda t, s, k: (0, t, 0)),
            pl.BlockSpec((N_CB, R), lambda t, s, k: (0, t)),
            pl.BlockSpec((1, 1, 1, 128), lambda t, s, k: (t, s, 0, 0)),
        ],
        out_shape=[
            jax.ShapeDtypeStruct((KC, ROWS, KW), f32),
            jax.ShapeDtypeStruct((N_CB, ROWS), jnp.int32),
            jax.ShapeDtypeStruct((N_TILES, N_CB, 1, 128), f32),
        ],
        scratch_shapes=[
            pltpu.VMEM((KC, R, KW), f32),
            pltpu.VMEM((R, CB_DIM), f32),
            pltpu.VMEM((KC, R, KW), f32),
        ],
        compiler_params=pltpu.CompilerParams(
            dimension_semantics=("arbitrary", "arbitrary", "arbitrary")),
    )(xr4, win4, b_in3, cbn_b, ct, cb_hi, cb_lo, cb_lolo, wout4, b_out4)

    z = z4.transpose(1, 0, 2).reshape(B, T, D).transpose(0, 2, 1)
    codes = codes.reshape(N_CB, B, T)
    commit = jnp.sum(loss_parts[:, :, 0, 0]) / jnp.float32(B * CB_DIM * T)
    return z, codes, commit, commit
